# baseline (speedup 1.0000x reference)
import functools

import jax
import jax.numpy as jnp
from jax import lax
from jax.experimental import pallas as pl
from jax.experimental.pallas import tpu as pltpu
from jax.experimental.pallas import tpu_sc as plsc

N = 50000
E = 800000
D = 64
H = 32

NPAD = 50048
ROWS_PER_TILE = NPAD // 16
NSUB = 10
CH = 128
SUP = NSUB * CH
NCHUNK = 40
NCH_TOT = 41
E3 = 16 * NCH_TOT * SUP
G = 5
LAG = 3
NSUB_B = 12
NCHUNK_B = 33
NCH_TOT_B = 34
G_B = 6
LAG_B = 5

_MESH = plsc.VectorSubcoreMesh(core_axis_name="c", subcore_axis_name="s")
_SC_PARAMS = pltpu.CompilerParams(use_tc_tiling_on_sc=False)


def _deg_body(col6, zeros1, degp, degS, colbuf, obuf, ssem, isem):
    c = lax.axis_index("c")
    sid = lax.axis_index("s")
    base = sid * ROWS_PER_TILE
    for i in range(CH // 16):
        obuf[pl.ds(i * 16, 16)] = jnp.ones((16,), jnp.float32)
    pltpu.sync_copy(zeros1.at[pl.ds(base, ROWS_PER_TILE)],
                    degS.at[pl.ds(base, ROWS_PER_TILE)])
    plsc.subcore_barrier()

    def wait_i():
        pltpu.make_async_copy(col6.at[sid, 0], colbuf.at[0], isem).wait()

    pltpu.sync_copy(col6.at[sid, c], colbuf.at[0])

    def chunk_body(k, _):
        ch = 2 * k + c
        p = k % 2
        pltpu.async_copy(col6.at[sid, ch + 2], colbuf.at[1 - p], isem)

        @pl.when(ch < NCH_TOT_B)
        def _():
            descs = [pltpu.async_copy(obuf, degS.at[colbuf.at[p, j]], ssem,
                                      add=True) for j in range(NSUB_B)]
            for d in descs:
                d.wait()
        wait_i()
        return ()

    lax.fori_loop(0, (NCH_TOT_B + 1) // 2, chunk_body, ())
    plsc.subcore_barrier()
    pltpu.sync_copy(degS.at[pl.ds(base, ROWS_PER_TILE)],
                    degp.at[c, pl.ds(base, ROWS_PER_TILE)])


_deg_kernel = pl.kernel(
    _deg_body,
    out_type=jax.ShapeDtypeStruct((2, NPAD), jnp.float32),
    mesh=_MESH,
    compiler_params=_SC_PARAMS,
    scratch_types=[
        pltpu.VMEM_SHARED((NPAD,), jnp.float32),
        pltpu.VMEM((2, NSUB_B, CH), jnp.int32),
        pltpu.VMEM((CH,), jnp.float32),
        pltpu.SemaphoreType.DMA,
        pltpu.SemaphoreType.DMA,
    ],
)


def _spmv_body(with_t, dims, col5, row5, yall, zeros2, *rest):
    NSUB_, NCHUNK_, G_, LAG_ = dims
    SLAG_ = G_ - LAG_
    if with_t:
        (zeros1, disp, zout, tp, Z, tS, colbuf, rowbuf, gbuf, dbuf,
         gsem, ssem, isem, dsem) = rest
    else:
        (zout, Z, colbuf, rowbuf, gbuf, gsem, ssem, isem) = rest
    c = lax.axis_index("c")
    sid = lax.axis_index("s")
    base = sid * ROWS_PER_TILE
    pltpu.sync_copy(zeros2.at[pl.ds(base, ROWS_PER_TILE)],
                    Z.at[pl.ds(base, ROWS_PER_TILE)])
    if with_t:
        pltpu.sync_copy(zeros1.at[pl.ds(base, ROWS_PER_TILE)],
                        tS.at[pl.ds(base, ROWS_PER_TILE)])
    plsc.subcore_barrier()
    ytab = yall.at[c]

    def wait_g():
        pltpu.make_async_copy(ytab.at[colbuf.at[0, 0]], gbuf.at[0],
                              gsem).wait()

    def wait_i():
        pltpu.make_async_copy(col5.at[sid, 0], colbuf.at[0], isem).wait()

    if with_t:
        def t_fire(p):
            for j in range(NSUB_):
                pltpu.async_copy(disp.at[colbuf.at[p, j]], dbuf.at[j], dsem)

        def t_drain(p):
            for j in range(NSUB_):
                pltpu.make_async_copy(disp.at[colbuf.at[p, 0]], dbuf.at[j],
                                      dsem).wait()
            td = [pltpu.async_copy(dbuf.at[j], tS.at[rowbuf.at[p, j]], dsem,
                                   add=True) for j in range(NSUB_)]
            for d in td:
                d.wait()

    pltpu.sync_copy(col5.at[sid, 0], colbuf.at[0])
    pltpu.sync_copy(row5.at[sid, 0], rowbuf.at[0])
    pltpu.async_copy(col5.at[sid, 1], colbuf.at[1], isem)
    pltpu.async_copy(row5.at[sid, 1], rowbuf.at[1], isem)
    if with_t:
        @pl.when(c == 0)
        def _():
            t_fire(0)
    for m in range(LAG_):
        pltpu.async_copy(ytab.at[colbuf.at[0, m]], gbuf.at[m % G_], gsem)
    sd = [None] * NSUB_
    for j in range(NSUB_):
        if j == NSUB_ - LAG_:
            wait_i()
            wait_i()
        if j >= SLAG_:
            sd[j - SLAG_].wait()
        if j + LAG_ < NSUB_:
            gidx = colbuf.at[0, j + LAG_]
        else:
            gidx = colbuf.at[1, j + LAG_ - NSUB_]
        pltpu.async_copy(ytab.at[gidx], gbuf.at[(j + LAG_) % G_], gsem)
        wait_g()
        sd[j] = pltpu.async_copy(gbuf.at[j % G_], Z.at[rowbuf.at[0, j]], ssem,
                                 add=True)
    for q in range(SLAG_):
        sd[NSUB_ - SLAG_ + q].wait()
    if with_t:
        @pl.when(c == 0)
        def _():
            t_drain(0)

    def chunk_body(ch, _):
        p = ch % 2
        pn = 1 - p
        if with_t:
            @pl.when(p == c)
            def _():
                t_fire(p)
        sd = [None] * NSUB_
        for j in range(NSUB_):
            if j == 0:
                pltpu.async_copy(col5.at[sid, ch + 1], colbuf.at[pn], isem)
                pltpu.async_copy(row5.at[sid, ch + 1], rowbuf.at[pn], isem)
            if j == NSUB_ - LAG_:
                wait_i()
                wait_i()
            if j >= SLAG_:
                sd[j - SLAG_].wait()
            if j + LAG_ < NSUB_:
                gidx = colbuf.at[p, j + LAG_]
            else:
                gidx = colbuf.at[pn, j + LAG_ - NSUB_]
            pltpu.async_copy(ytab.at[gidx], gbuf.at[(j + LAG_) % G_], gsem)
            wait_g()
            sd[j] = pltpu.async_copy(gbuf.at[j % G_], Z.at[rowbuf.at[p, j]],
                                     ssem, add=True)
        for q in range(SLAG_):
            sd[NSUB_ - SLAG_ + q].wait()
        if with_t:
            @pl.when(p == c)
            def _():
                t_drain(p)
        return ()

    lax.fori_loop(1, NCHUNK_, chunk_body, ())

    for _ in range(LAG_):
        wait_g()
    plsc.subcore_barrier()
    pltpu.sync_copy(Z.at[pl.ds(base, ROWS_PER_TILE)],
                    zout.at[c, pl.ds(base, ROWS_PER_TILE)])
    if with_t:
        pltpu.sync_copy(tS.at[pl.ds(base, ROWS_PER_TILE)],
                        tp.at[c, pl.ds(base, ROWS_PER_TILE)])


_spmv_kernel = pl.kernel(
    functools.partial(_spmv_body, False, (NSUB_B, NCHUNK_B, G_B, LAG_B)),
    out_type=jax.ShapeDtypeStruct((2, NPAD, H), jnp.float32),
    mesh=_MESH,
    compiler_params=_SC_PARAMS,
    scratch_types=[
        pltpu.VMEM_SHARED((NPAD, H), jnp.float32),
        pltpu.VMEM((2, NSUB_B, CH), jnp.int32),
        pltpu.VMEM((2, NSUB_B, CH), jnp.int32),
        pltpu.VMEM((G_B, CH, H), jnp.float32),
        pltpu.SemaphoreType.DMA,
        pltpu.SemaphoreType.DMA,
        pltpu.SemaphoreType.DMA,
    ],
)


_BLK = 3128
_GRID = NPAD // _BLK


def _d0_body(degp, emb, diso, y0):
    deg = degp[0] + degp[1]
    dis = jnp.where(deg > 0, lax.rsqrt(deg), 0.0)
    diso[...] = dis
    y = dis * emb[...]
    y0[0] = y[:, :H]
    y0[1] = y[:, H:]


def _tc_d0(degp2, emb):
    return pl.pallas_call(
        _d0_body,
        grid=(_GRID,),
        in_specs=[
            pl.BlockSpec((2, _BLK, 1), lambda i: (0, i, 0)),
            pl.BlockSpec((_BLK, D), lambda i: (i, 0)),
        ],
        out_specs=[
            pl.BlockSpec((_BLK, 1), lambda i: (i, 0)),
            pl.BlockSpec((2, _BLK, H), lambda i: (0, i, 0)),
        ],
        out_shape=[
            jax.ShapeDtypeStruct((NPAD, 1), jnp.float32),
            jax.ShapeDtypeStruct((2, NPAD, H), jnp.float32),
        ],
    )(degp2, emb)


def _agg(x, zl, zh, dis, w1, w2):
    Z = jnp.concatenate([zl[0], zh[0]], axis=1)
    S = dis * Z
    agg = (lax.dot_general(S, w1[...], (((1,), (1,)), ((), ())),
                           preferred_element_type=jnp.float32)
           + lax.dot_general(x * S, w2[...], (((1,), (1,)), ((), ())),
                             preferred_element_type=jnp.float32))
    return jnp.where(agg >= 0, agg, 0.2 * agg)


def _layer_body_first(emb, zl, zh, dis4, w1, w2, bufo, yo):
    dis = dis4[...]
    xb = emb[...]
    xn = _agg(xb, zl, zh, dis, w1, w2)
    bufo[...] = jnp.concatenate([xb, xn], axis=1)
    yn = dis * xn
    yo[0] = yn[:, :H]
    yo[1] = yn[:, H:]


def _layer_body_mid(buf, zl, zh, dis4, w1, w2, xo, yo):
    dis = dis4[...]
    xn = _agg(buf[:, D:], zl, zh, dis, w1, w2)
    xo[...] = xn
    yn = dis * xn
    yo[0] = yn[:, :H]
    yo[1] = yn[:, H:]


def _layer_body_last(x2, zl, zh, dis4, w1, w2, bufin, bufo):
    xb = x2[...]
    xn = _agg(xb, zl, zh, dis4[...], w1, w2)
    bufo[...] = jnp.concatenate([xb, xn], axis=1)


def _buf_spec(cb):
    return pl.BlockSpec((_BLK, 2 * D), lambda i, _c=cb: (i, _c))


_vec_spec = pl.BlockSpec((_BLK, 1), lambda i: (i, 0))
_vec2_spec = pl.BlockSpec((2, _BLK, 1), lambda i: (0, i, 0))
_zl_spec = pl.BlockSpec((1, _BLK, H), lambda i: (0, i, 0))
_zh_spec = pl.BlockSpec((1, _BLK, H), lambda i: (1, i, 0))
_w_spec = pl.BlockSpec((D, D), lambda i: (0, 0))
_b_spec = pl.BlockSpec((1, D), lambda i: (0, 0))
_y_spec = pl.BlockSpec((2, _BLK, H), lambda i: (0, i, 0))
_BUF_SHAPE = jax.ShapeDtypeStruct((N, 4 * D), jnp.float32)


def _tc_first(emb, zout, dis4, w1, w2):
    return pl.pallas_call(
        _layer_body_first,
        grid=(_GRID,),
        in_specs=[pl.BlockSpec((_BLK, D), lambda i: (i, 0)), _zl_spec,
                  _zh_spec, _vec_spec, _w_spec, _w_spec],
        out_specs=[_buf_spec(0), _y_spec],
        out_shape=[
            _BUF_SHAPE,
            jax.ShapeDtypeStruct((2, NPAD, H), jnp.float32),
        ],
    )(emb, zout, zout, dis4, w1, w2)


def _tc_mid(buf, zout, dis4, w1, w2):
    return pl.pallas_call(
        _layer_body_mid,
        grid=(_GRID,),
        in_specs=[_buf_spec(0), _zl_spec, _zh_spec, _vec_spec,
                  _w_spec, _w_spec],
        out_specs=[pl.BlockSpec((_BLK, D), lambda i: (i, 0)), _y_spec],
        out_shape=[
            jax.ShapeDtypeStruct((NPAD, D), jnp.float32),
            jax.ShapeDtypeStruct((2, NPAD, H), jnp.float32),
        ],
    )(buf, zout, zout, dis4, w1, w2)


def _tc_last(x2, zout, dis4, w1, w2, buf):
    return pl.pallas_call(
        _layer_body_last,
        grid=(_GRID,),
        in_specs=[pl.BlockSpec((_BLK, D), lambda i: (i, 0)), _zl_spec,
                  _zh_spec, _vec_spec, _w_spec, _w_spec,
                  pl.BlockSpec((8, 128), lambda i: (0, 0))],
        out_specs=_buf_spec(1),
        out_shape=_BUF_SHAPE,
        input_output_aliases={6: 0},
    )(x2, zout, zout, dis4, w1, w2, buf)


def kernel(edge_index, emb, W1_0, b1_0, W2_0, b2_0, W1_1, b1_1, W2_1, b2_1,
           W1_2, b1_2, W2_2, b2_2):
    row = edge_index[0]
    col = edge_index[1]
    eptb = NCH_TOT_B * NSUB_B * CH
    col5b = jnp.pad(col.reshape(16, E // 16), ((0, 0), (0, eptb - E // 16)),
                    constant_values=N).reshape(16, NCH_TOT_B, NSUB_B, CH)
    row5b = jnp.pad(row.reshape(16, E // 16), ((0, 0), (0, eptb - E // 16)),
                    constant_values=N).reshape(16, NCH_TOT_B, NSUB_B, CH)
    col6 = jnp.pad(col.reshape(16, E // 16),
                   ((0, 0), (0, eptb + 2 * NSUB_B * CH - E // 16)),
                   constant_values=N).reshape(16, NCH_TOT_B + 2, NSUB_B, CH)
    zeros1 = jnp.zeros((NPAD,), jnp.float32)
    zeros2 = jnp.zeros((NPAD, H), jnp.float32)

    degp = _deg_kernel(col6, zeros1)
    dis4, y0 = _tc_d0(degp.reshape(2, NPAD, 1), emb)

    zout1 = _spmv_kernel(col5b, row5b, y0, zeros2)
    buf, y1 = _tc_first(emb, zout1, dis4, W1_0, W2_0)

    zout2 = _spmv_kernel(col5b, row5b, y1, zeros2)
    x2, y2 = _tc_mid(buf, zout2, dis4, W1_1, W2_1)

    zout3 = _spmv_kernel(col5b, row5b, y2, zeros2)
    return _tc_last(x2, zout3, dis4, W1_2, W2_2, buf)


# speedup vs baseline: 1.0337x; 1.0337x over previous
import functools

import jax
import jax.numpy as jnp
from jax import lax
from jax.experimental import pallas as pl
from jax.experimental.pallas import tpu as pltpu
from jax.experimental.pallas import tpu_sc as plsc

N = 50000
E = 800000
D = 64
H = 32

NPAD = 50048
ROWS_PER_TILE = NPAD // 16
NSUB = 10
CH = 128
SUP = NSUB * CH
NCHUNK = 40
NCH_TOT = 41
E3 = 16 * NCH_TOT * SUP
G = 5
LAG = 3
NSUB_B = 12
NCHUNK_B = 33
NCH_TOT_B = 34
G_B = 6
LAG_B = 4

_MESH = plsc.VectorSubcoreMesh(core_axis_name="c", subcore_axis_name="s")
_SC_PARAMS = pltpu.CompilerParams(use_tc_tiling_on_sc=False)


def _deg_body(col6, zeros1, degp, degS, colbuf, obuf, ssem, isem):
    c = lax.axis_index("c")
    sid = lax.axis_index("s")
    base = sid * ROWS_PER_TILE
    for i in range(CH // 16):
        obuf[pl.ds(i * 16, 16)] = jnp.ones((16,), jnp.float32)
    pltpu.sync_copy(zeros1.at[pl.ds(base, ROWS_PER_TILE)],
                    degS.at[pl.ds(base, ROWS_PER_TILE)])
    plsc.subcore_barrier()

    def wait_i():
        pltpu.make_async_copy(col6.at[sid, 0], colbuf.at[0], isem).wait()

    pltpu.sync_copy(col6.at[sid, c], colbuf.at[0])

    def chunk_body(k, _):
        ch = 2 * k + c
        p = k % 2
        pltpu.async_copy(col6.at[sid, ch + 2], colbuf.at[1 - p], isem)

        @pl.when(ch < NCH_TOT_B)
        def _():
            descs = [pltpu.async_copy(obuf, degS.at[colbuf.at[p, j]], ssem,
                                      add=True) for j in range(NSUB_B)]
            for d in descs:
                d.wait()
        wait_i()
        return ()

    lax.fori_loop(0, (NCH_TOT_B + 1) // 2, chunk_body, ())
    plsc.subcore_barrier()
    pltpu.sync_copy(degS.at[pl.ds(base, ROWS_PER_TILE)],
                    degp.at[c, pl.ds(base, ROWS_PER_TILE)])


_deg_kernel = pl.kernel(
    _deg_body,
    out_type=jax.ShapeDtypeStruct((2, NPAD), jnp.float32),
    mesh=_MESH,
    compiler_params=_SC_PARAMS,
    scratch_types=[
        pltpu.VMEM_SHARED((NPAD,), jnp.float32),
        pltpu.VMEM((2, NSUB_B, CH), jnp.int32),
        pltpu.VMEM((CH,), jnp.float32),
        pltpu.SemaphoreType.DMA,
        pltpu.SemaphoreType.DMA,
    ],
)


def _spmv_body(with_t, dims, col5, row5, yall, zeros2, *rest):
    NSUB_, NCHUNK_, G_, LAG_ = dims
    if with_t:
        (zeros1, disp, zout, tp, Z, tS, colbuf, rowbuf, gbuf, dbuf,
         gsem, ssem, isem, dsem) = rest
    else:
        (zout, Z, colbuf, rowbuf, gbuf, gsem, ssem, isem) = rest
    c = lax.axis_index("c")
    sid = lax.axis_index("s")
    base = sid * ROWS_PER_TILE
    pltpu.sync_copy(zeros2.at[pl.ds(base, ROWS_PER_TILE)],
                    Z.at[pl.ds(base, ROWS_PER_TILE)])
    if with_t:
        pltpu.sync_copy(zeros1.at[pl.ds(base, ROWS_PER_TILE)],
                        tS.at[pl.ds(base, ROWS_PER_TILE)])
    plsc.subcore_barrier()
    ytab = yall.at[c]

    def wait_g():
        pltpu.make_async_copy(ytab.at[colbuf.at[0, 0]], gbuf.at[0],
                              gsem).wait()

    def wait_i():
        pltpu.make_async_copy(col5.at[sid, 0], colbuf.at[0], isem).wait()

    if with_t:
        def t_fire(p):
            for j in range(NSUB_):
                pltpu.async_copy(disp.at[colbuf.at[p, j]], dbuf.at[j], dsem)

        def t_drain(p):
            for j in range(NSUB_):
                pltpu.make_async_copy(disp.at[colbuf.at[p, 0]], dbuf.at[j],
                                      dsem).wait()
            td = [pltpu.async_copy(dbuf.at[j], tS.at[rowbuf.at[p, j]], dsem,
                                   add=True) for j in range(NSUB_)]
            for d in td:
                d.wait()

    pltpu.sync_copy(col5.at[sid, 0], colbuf.at[0])
    pltpu.sync_copy(row5.at[sid, 0], rowbuf.at[0])
    pltpu.async_copy(col5.at[sid, 1], colbuf.at[1], isem)
    pltpu.async_copy(row5.at[sid, 1], rowbuf.at[1], isem)
    if with_t:
        @pl.when(c == 0)
        def _():
            t_fire(0)
    for m in range(LAG_):
        pltpu.async_copy(ytab.at[colbuf.at[0, m]], gbuf.at[m % G_], gsem)
    sd = [None] * NSUB_
    for j in range(NSUB_):
        if j == NSUB_ - LAG_:
            wait_i()
            wait_i()
        if j >= 2:
            sd[j - 2].wait()
        if j + LAG_ < NSUB_:
            gidx = colbuf.at[0, j + LAG_]
        else:
            gidx = colbuf.at[1, j + LAG_ - NSUB_]
        pltpu.async_copy(ytab.at[gidx], gbuf.at[(j + LAG_) % G_], gsem)
        wait_g()
        sd[j] = pltpu.async_copy(gbuf.at[j % G_], Z.at[rowbuf.at[0, j]], ssem,
                                 add=True)
    sd[NSUB_ - 2].wait()
    sd[NSUB_ - 1].wait()
    if with_t:
        @pl.when(c == 0)
        def _():
            t_drain(0)

    def chunk_body(ch, _):
        p = ch % 2
        pn = 1 - p
        if with_t:
            @pl.when(p == c)
            def _():
                t_fire(p)
        sd = [None] * NSUB_
        for j in range(NSUB_):
            if j == 0:
                pltpu.async_copy(col5.at[sid, ch + 1], colbuf.at[pn], isem)
                pltpu.async_copy(row5.at[sid, ch + 1], rowbuf.at[pn], isem)
            if j == NSUB_ - LAG_:
                wait_i()
                wait_i()
            if j >= 2:
                sd[j - 2].wait()
            if j + LAG_ < NSUB_:
                gidx = colbuf.at[p, j + LAG_]
            else:
                gidx = colbuf.at[pn, j + LAG_ - NSUB_]
            pltpu.async_copy(ytab.at[gidx], gbuf.at[(j + LAG_) % G_], gsem)
            wait_g()
            sd[j] = pltpu.async_copy(gbuf.at[j % G_], Z.at[rowbuf.at[p, j]],
                                     ssem, add=True)
        sd[NSUB_ - 2].wait()
        sd[NSUB_ - 1].wait()
        if with_t:
            @pl.when(p == c)
            def _():
                t_drain(p)
        return ()

    lax.fori_loop(1, NCHUNK_, chunk_body, ())

    for _ in range(LAG_):
        wait_g()
    plsc.subcore_barrier()
    pltpu.sync_copy(Z.at[pl.ds(base, ROWS_PER_TILE)],
                    zout.at[c, pl.ds(base, ROWS_PER_TILE)])
    if with_t:
        pltpu.sync_copy(tS.at[pl.ds(base, ROWS_PER_TILE)],
                        tp.at[c, pl.ds(base, ROWS_PER_TILE)])


_spmv_kernel = pl.kernel(
    functools.partial(_spmv_body, False, (NSUB_B, NCHUNK_B, G_B, LAG_B)),
    out_type=jax.ShapeDtypeStruct((2, NPAD, H), jnp.float32),
    mesh=_MESH,
    compiler_params=_SC_PARAMS,
    scratch_types=[
        pltpu.VMEM_SHARED((NPAD, H), jnp.float32),
        pltpu.VMEM((2, NSUB_B, CH), jnp.int32),
        pltpu.VMEM((2, NSUB_B, CH), jnp.int32),
        pltpu.VMEM((G_B, CH, H), jnp.float32),
        pltpu.SemaphoreType.DMA,
        pltpu.SemaphoreType.DMA,
        pltpu.SemaphoreType.DMA,
    ],
)


_BLK = 3128
_GRID = NPAD // _BLK


def _d0_body(degp, emb, diso, y0):
    deg = degp[0] + degp[1]
    dis = jnp.where(deg > 0, lax.rsqrt(deg), 0.0)
    diso[...] = dis
    y = dis * emb[...]
    y0[0] = y[:, :H]
    y0[1] = y[:, H:]


def _tc_d0(degp2, emb):
    return pl.pallas_call(
        _d0_body,
        grid=(_GRID,),
        in_specs=[
            pl.BlockSpec((2, _BLK, 1), lambda i: (0, i, 0)),
            pl.BlockSpec((_BLK, D), lambda i: (i, 0)),
        ],
        out_specs=[
            pl.BlockSpec((_BLK, 1), lambda i: (i, 0)),
            pl.BlockSpec((2, _BLK, H), lambda i: (0, i, 0)),
        ],
        out_shape=[
            jax.ShapeDtypeStruct((NPAD, 1), jnp.float32),
            jax.ShapeDtypeStruct((2, NPAD, H), jnp.float32),
        ],
    )(degp2, emb)


def _agg(x, zl, zh, dis, w1, w2):
    Z = jnp.concatenate([zl[0], zh[0]], axis=1)
    S = dis * Z
    agg = (lax.dot_general(S, w1[...], (((1,), (1,)), ((), ())),
                           preferred_element_type=jnp.float32)
           + lax.dot_general(x * S, w2[...], (((1,), (1,)), ((), ())),
                             preferred_element_type=jnp.float32))
    return jnp.where(agg >= 0, agg, 0.2 * agg)


def _layer_body_first(emb, zl, zh, dis4, w1, w2, bufo, yo):
    dis = dis4[...]
    xb = emb[...]
    xn = _agg(xb, zl, zh, dis, w1, w2)
    bufo[...] = jnp.concatenate([xb, xn], axis=1)
    yn = dis * xn
    yo[0] = yn[:, :H]
    yo[1] = yn[:, H:]


def _layer_body_mid(buf, zl, zh, dis4, w1, w2, xo, yo):
    dis = dis4[...]
    xn = _agg(buf[:, D:], zl, zh, dis, w1, w2)
    xo[...] = xn
    yn = dis * xn
    yo[0] = yn[:, :H]
    yo[1] = yn[:, H:]


def _layer_body_last(x2, zl, zh, dis4, w1, w2, bufin, bufo):
    xb = x2[...]
    xn = _agg(xb, zl, zh, dis4[...], w1, w2)
    bufo[...] = jnp.concatenate([xb, xn], axis=1)


def _buf_spec(cb):
    return pl.BlockSpec((_BLK, 2 * D), lambda i, _c=cb: (i, _c))


_vec_spec = pl.BlockSpec((_BLK, 1), lambda i: (i, 0))
_vec2_spec = pl.BlockSpec((2, _BLK, 1), lambda i: (0, i, 0))
_zl_spec = pl.BlockSpec((1, _BLK, H), lambda i: (0, i, 0))
_zh_spec = pl.BlockSpec((1, _BLK, H), lambda i: (1, i, 0))
_w_spec = pl.BlockSpec((D, D), lambda i: (0, 0))
_b_spec = pl.BlockSpec((1, D), lambda i: (0, 0))
_y_spec = pl.BlockSpec((2, _BLK, H), lambda i: (0, i, 0))
_BUF_SHAPE = jax.ShapeDtypeStruct((N, 4 * D), jnp.float32)


def _tc_first(emb, zout, dis4, w1, w2):
    return pl.pallas_call(
        _layer_body_first,
        grid=(_GRID,),
        in_specs=[pl.BlockSpec((_BLK, D), lambda i: (i, 0)), _zl_spec,
                  _zh_spec, _vec_spec, _w_spec, _w_spec],
        out_specs=[_buf_spec(0), _y_spec],
        out_shape=[
            _BUF_SHAPE,
            jax.ShapeDtypeStruct((2, NPAD, H), jnp.float32),
        ],
    )(emb, zout, zout, dis4, w1, w2)


def _tc_mid(buf, zout, dis4, w1, w2):
    return pl.pallas_call(
        _layer_body_mid,
        grid=(_GRID,),
        in_specs=[_buf_spec(0), _zl_spec, _zh_spec, _vec_spec,
                  _w_spec, _w_spec],
        out_specs=[pl.BlockSpec((_BLK, D), lambda i: (i, 0)), _y_spec],
        out_shape=[
            jax.ShapeDtypeStruct((NPAD, D), jnp.float32),
            jax.ShapeDtypeStruct((2, NPAD, H), jnp.float32),
        ],
    )(buf, zout, zout, dis4, w1, w2)


def _tc_last(x2, zout, dis4, w1, w2, buf):
    return pl.pallas_call(
        _layer_body_last,
        grid=(_GRID,),
        in_specs=[pl.BlockSpec((_BLK, D), lambda i: (i, 0)), _zl_spec,
                  _zh_spec, _vec_spec, _w_spec, _w_spec,
                  pl.BlockSpec((8, 128), lambda i: (0, 0))],
        out_specs=_buf_spec(1),
        out_shape=_BUF_SHAPE,
        input_output_aliases={6: 0},
    )(x2, zout, zout, dis4, w1, w2, buf)


def kernel(edge_index, emb, W1_0, b1_0, W2_0, b2_0, W1_1, b1_1, W2_1, b2_1,
           W1_2, b1_2, W2_2, b2_2):
    row = edge_index[0]
    col = edge_index[1]
    eptb = NCH_TOT_B * NSUB_B * CH
    col5b = jnp.pad(col.reshape(16, E // 16), ((0, 0), (0, eptb - E // 16)),
                    constant_values=N).reshape(16, NCH_TOT_B, NSUB_B, CH)
    row5b = jnp.pad(row.reshape(16, E // 16), ((0, 0), (0, eptb - E // 16)),
                    constant_values=N).reshape(16, NCH_TOT_B, NSUB_B, CH)
    col6 = jnp.pad(col.reshape(16, E // 16),
                   ((0, 0), (0, eptb + 2 * NSUB_B * CH - E // 16)),
                   constant_values=N).reshape(16, NCH_TOT_B + 2, NSUB_B, CH)
    zeros1 = jnp.zeros((NPAD,), jnp.float32)
    zeros2 = jnp.zeros((NPAD, H), jnp.float32)

    degp = _deg_kernel(col6, zeros1)
    dis4, y0 = _tc_d0(degp.reshape(2, NPAD, 1), emb)

    zout1 = _spmv_kernel(col5b, row5b, y0, zeros2)
    buf, y1 = _tc_first(emb, zout1, dis4, W1_0, W2_0)

    zout2 = _spmv_kernel(col5b, row5b, y1, zeros2)
    x2, y2 = _tc_mid(buf, zout2, dis4, W1_1, W2_1)

    zout3 = _spmv_kernel(col5b, row5b, y2, zeros2)
    return _tc_last(x2, zout3, dis4, W1_2, W2_2, buf)
